# initial kernel scaffold (unmeasured)
import jax
import jax.numpy as jnp
from jax import lax
from jax.experimental import pallas as pl
from jax.experimental.pallas import tpu as pltpu

N_DEV = 16


def kernel(x, w_mat):
    p = jnp.dot(x, w_mat, preferred_element_type=jnp.float32)
    m, n = p.shape
    cm = m // N_DEV

    def body(p_ref, out_ref, comm, chunk, send_sems, recv_sems, copy_sem):
        d = lax.axis_index("i")
        left = jnp.mod(d - 1, N_DEV)
        right = jnp.mod(d + 1, N_DEV)

        barrier = pltpu.get_barrier_semaphore()
        for nbr in (left, right):
            pl.semaphore_signal(
                barrier, inc=1, device_id=(nbr,),
                device_id_type=pl.DeviceIdType.MESH,
            )
        pl.semaphore_wait(barrier, 2)

        def load_chunk(c, dst):
            cp = pltpu.make_async_copy(
                p_ref.at[pl.ds(c * cm, cm), :], dst, copy_sem)
            cp.start()
            cp.wait()

        def store_chunk(src, c):
            cp = pltpu.make_async_copy(
                src, out_ref.at[pl.ds(c * cm, cm), :], copy_sem)
            cp.start()
            cp.wait()

        def hop(s_slot, r_slot):
            rdma = pltpu.make_async_remote_copy(
                src_ref=comm.at[s_slot],
                dst_ref=comm.at[r_slot],
                send_sem=send_sems.at[s_slot],
                recv_sem=recv_sems.at[r_slot],
                device_id=(right,),
                device_id_type=pl.DeviceIdType.MESH,
            )
            rdma.start()
            return rdma

        load_chunk(d, comm.at[0])
        for h in range(N_DEV - 1):
            s_slot, r_slot = h % 2, (h + 1) % 2
            rdma = hop(s_slot, r_slot)
            load_chunk(jnp.mod(d - 1 - h, N_DEV), chunk)
            rdma.wait()
            comm[r_slot] = comm[r_slot] + chunk[...]

        own_slot = (N_DEV - 1) % 2
        own_c = jnp.mod(d + 1, N_DEV)
        amax = jnp.max(comm[own_slot])
        store_chunk(comm.at[own_slot], own_c)

        for g in range(N_DEV - 1):
            s_slot, r_slot = (g + 1) % 2, g % 2
            rdma = hop(s_slot, r_slot)
            rdma.wait()
            c = jnp.mod(d - g, N_DEV)
            amax = jnp.maximum(amax, jnp.max(comm[r_slot]))
            store_chunk(comm.at[r_slot], c)

        amax = jnp.maximum(amax, 0.0)
        scale = jnp.maximum(amax, 1e-20) / 127.0
        for c in range(N_DEV):
            cp = pltpu.make_async_copy(
                out_ref.at[pl.ds(c * cm, cm), :], chunk, copy_sem)
            cp.start()
            cp.wait()
            v = jnp.maximum(chunk[...], 0.0)
            q = jnp.clip(jnp.round(v / scale), -127.0, 127.0)
            chunk[...] = q * scale
            cp = pltpu.make_async_copy(
                chunk, out_ref.at[pl.ds(c * cm, cm), :], copy_sem)
            cp.start()
            cp.wait()

    return pl.pallas_call(
        body,
        out_shape=jax.ShapeDtypeStruct((m, n), jnp.float32),
        in_specs=[pl.BlockSpec(memory_space=pltpu.ANY)],
        out_specs=pl.BlockSpec(memory_space=pltpu.ANY),
        scratch_shapes=[
            pltpu.VMEM((2, cm, n), jnp.float32),
            pltpu.VMEM((cm, n), jnp.float32),
            pltpu.SemaphoreType.DMA((2,)),
            pltpu.SemaphoreType.DMA((2,)),
            pltpu.SemaphoreType.DMA,
        ],
        compiler_params=pltpu.CompilerParams(collective_id=0),
    )(p)


# baseline (device time: 3114817 ns/iter reference)
import jax
import jax.numpy as jnp
from jax import lax
from jax.experimental import pallas as pl
from jax.experimental.pallas import tpu as pltpu

N_DEV = 16


def kernel(x, w_mat):
    p = jnp.dot(x, w_mat, preferred_element_type=jnp.float32)
    m, n = p.shape
    cm = m // N_DEV

    def body(p_ref, out_ref, comm, chunk, amax_ref, send_sems, recv_sems,
             copy_sem):
        d = lax.axis_index("i")
        left = jnp.mod(d - 1, N_DEV)
        right = jnp.mod(d + 1, N_DEV)

        barrier = pltpu.get_barrier_semaphore()
        for nbr in (left, right):
            pl.semaphore_signal(
                barrier, inc=1, device_id=(nbr,),
                device_id_type=pl.DeviceIdType.MESH,
            )
        pl.semaphore_wait(barrier, 2)

        def load_chunk(c, dst):
            cp = pltpu.make_async_copy(
                p_ref.at[pl.ds(c * cm, cm), :], dst, copy_sem)
            cp.start()
            cp.wait()

        def store_chunk(src, c):
            cp = pltpu.make_async_copy(
                src, out_ref.at[pl.ds(c * cm, cm), :], copy_sem)
            cp.start()
            cp.wait()

        def hop(s_slot, r_slot):
            rdma = pltpu.make_async_remote_copy(
                src_ref=comm.at[s_slot],
                dst_ref=comm.at[r_slot],
                send_sem=send_sems.at[s_slot],
                recv_sem=recv_sems.at[r_slot],
                device_id=(right,),
                device_id_type=pl.DeviceIdType.MESH,
            )
            rdma.start()
            return rdma

        load_chunk(d, comm.at[0])

        def rs_step(h, _):
            s_slot = jnp.mod(h, 2)
            r_slot = jnp.mod(h + 1, 2)
            rdma = hop(s_slot, r_slot)
            load_chunk(jnp.mod(d - 1 - h, N_DEV), chunk)
            rdma.wait()
            comm[r_slot] = comm[r_slot] + chunk[...]
            return 0

        lax.fori_loop(0, N_DEV - 1, rs_step, 0)

        own_slot = (N_DEV - 1) % 2
        own_c = jnp.mod(d + 1, N_DEV)
        amax_ref[0, 0] = jnp.max(comm[own_slot])
        store_chunk(comm.at[own_slot], own_c)

        def ag_step(g, _):
            s_slot = jnp.mod(g + 1, 2)
            r_slot = jnp.mod(g, 2)
            rdma = hop(s_slot, r_slot)
            rdma.wait()
            c = jnp.mod(d - g, N_DEV)
            amax_ref[0, 0] = jnp.maximum(amax_ref[0, 0], jnp.max(comm[r_slot]))
            store_chunk(comm.at[r_slot], c)
            return 0

        lax.fori_loop(0, N_DEV - 1, ag_step, 0)

        amax = jnp.maximum(amax_ref[0, 0], 0.0)
        scale = jnp.maximum(amax, 1e-20) / 127.0

        def ep_step(c, _):
            cp = pltpu.make_async_copy(
                out_ref.at[pl.ds(c * cm, cm), :], chunk, copy_sem)
            cp.start()
            cp.wait()
            v = jnp.maximum(chunk[...], 0.0)
            q = jnp.clip(jnp.round(v / scale), -127.0, 127.0)
            chunk[...] = q * scale
            cp = pltpu.make_async_copy(
                chunk, out_ref.at[pl.ds(c * cm, cm), :], copy_sem)
            cp.start()
            cp.wait()
            return 0

        lax.fori_loop(0, N_DEV, ep_step, 0)

    return pl.pallas_call(
        body,
        out_shape=jax.ShapeDtypeStruct((m, n), jnp.float32),
        in_specs=[pl.BlockSpec(memory_space=pltpu.MemorySpace.HBM)],
        out_specs=pl.BlockSpec(memory_space=pltpu.MemorySpace.HBM),
        scratch_shapes=[
            pltpu.VMEM((2, cm, n), jnp.float32),
            pltpu.VMEM((cm, n), jnp.float32),
            pltpu.SMEM((1, 1), jnp.float32),
            pltpu.SemaphoreType.DMA((2,)),
            pltpu.SemaphoreType.DMA((2,)),
            pltpu.SemaphoreType.DMA,
        ],
        compiler_params=pltpu.CompilerParams(collective_id=0),
    )(p)
